# baseline (speedup 1.0000x reference)
import functools

import jax
import jax.numpy as jnp
from jax import lax
from jax.experimental import pallas as pl
from jax.experimental.pallas import tpu as pltpu
from jax.experimental.pallas import tpu_sc as plsc

D = 128
TEMP_INV = 10.0
EPS = 1e-8
NC = 2
NS = 16
NW = NC * NS
L = 16
CHUNK = 128


def _fast_rsqrt(x):
    i = lax.bitcast_convert_type(x, jnp.int32)
    i = jnp.int32(0x5F3759DF) - lax.shift_right_arithmetic(i, 1)
    y = lax.bitcast_convert_type(i, jnp.float32)
    for _ in range(3):
        y = y * (1.5 - 0.5 * x * y * y)
    return y


def _make_sc_kernel(n_pairs, n_pos):
    assert n_pairs % (NW * CHUNK) == 0
    ppt = n_pairs // NW
    nch = ppt // CHUNK
    mesh = plsc.VectorSubcoreMesh(core_axis_name="c", subcore_axis_name="s")

    @functools.partial(
        pl.kernel,
        mesh=mesh,
        compiler_params=pltpu.CompilerParams(
            needs_layout_passes=False, use_tc_tiling_on_sc=False),
        out_type=[
            jax.ShapeDtypeStruct((NW, L), jnp.float32),
            jax.ShapeDtypeStruct((NW, L), jnp.float32),
        ],
        scratch_types=[
            pltpu.VMEM((ppt,), jnp.int32),
            pltpu.VMEM((ppt,), jnp.int32),
            pltpu.VMEM((CHUNK, D), jnp.float32),
            pltpu.VMEM((CHUNK, D), jnp.float32),
            pltpu.VMEM((L,), jnp.float32),
            pltpu.VMEM((L,), jnp.float32),
            pltpu.SemaphoreType.DMA,
            pltpu.SemaphoreType.DMA,
        ],
    )
    def sc_kernel(z_hbm, ii_hbm, jj_hbm, oexp_hbm, opos_hbm,
                  ii_v, jj_v, rows_i, rows_j,
                  accexp, accpos, sem_i, sem_j):
        wid = lax.axis_index("s") * NC + lax.axis_index("c")
        base = wid * ppt
        pltpu.sync_copy(ii_hbm.at[pl.ds(base, ppt)], ii_v)
        pltpu.sync_copy(jj_hbm.at[pl.ds(base, ppt)], jj_v)
        accexp[...] = jnp.zeros((L,), jnp.float32)
        accpos[...] = jnp.zeros((L,), jnp.float32)
        lane = lax.broadcasted_iota(jnp.int32, (L,), 0)

        def chunk_body(ch, _):
            off = ch * CHUNK
            cp_i = pltpu.async_copy(
                z_hbm.at[ii_v.at[pl.ds(off, CHUNK)]], rows_i, sem_i)
            cp_j = pltpu.async_copy(
                z_hbm.at[jj_v.at[pl.ds(off, CHUNK)]], rows_j, sem_j)
            cp_i.wait()
            cp_j.wait()

            def group_body(g, _):
                pvec = g * L + lane
                ab_vec = jnp.zeros((L,), jnp.float32)
                aa_vec = jnp.zeros((L,), jnp.float32)
                bb_vec = jnp.zeros((L,), jnp.float32)
                for e in range(D):
                    evec = jnp.full((L,), e, jnp.int32)
                    av = plsc.load_gather(rows_i, [pvec, evec])
                    bv = plsc.load_gather(rows_j, [pvec, evec])
                    ab_vec = ab_vec + av * bv
                    aa_vec = aa_vec + av * av
                    bb_vec = bb_vec + bv * bv
                na = aa_vec * _fast_rsqrt(aa_vec)
                nb = bb_vec * _fast_rsqrt(bb_vec)
                denom = jnp.maximum(na, EPS) * jnp.maximum(nb, EPS)
                logit = (ab_vec / denom) * TEMP_INV
                accexp[...] += jnp.exp(logit - 10.0)
                gidx = base + off + g * L + lane
                accpos[...] += jnp.where(gidx < n_pos, logit, 0.0)
                return 0

            lax.fori_loop(0, CHUNK // L, group_body, 0)
            return 0

        lax.fori_loop(0, nch, chunk_body, 0)
        pltpu.sync_copy(accexp, oexp_hbm.at[wid])
        pltpu.sync_copy(accpos, opos_hbm.at[wid])

    return sc_kernel


def kernel(z_all, pos_pairs, neg_pairs):
    n_pos = pos_pairs.shape[0]
    pairs = jnp.concatenate([pos_pairs, neg_pairs], axis=0)
    ii = pairs[:, 0]
    jj = pairs[:, 1]
    sc = _make_sc_kernel(pairs.shape[0], n_pos)
    part_exp, part_pos = sc(z_all, ii, jj)
    lse = 10.0 + jnp.log(jnp.sum(part_exp))
    return lse - jnp.sum(part_pos) / n_pos


# speedup vs baseline: 4.3187x; 4.3187x over previous
import functools

import jax
import jax.numpy as jnp
from jax import lax
from jax.experimental import pallas as pl
from jax.experimental.pallas import tpu as pltpu
from jax.experimental.pallas import tpu_sc as plsc

D = 128
TEMP_INV = 10.0
EPS = 1e-8
NC = 2
NS = 16
NW = NC * NS
L = 16
CHUNK = 128


def _fast_rsqrt(x):
    i = lax.bitcast_convert_type(x, jnp.int32)
    i = jnp.int32(0x5F3759DF) - lax.shift_right_arithmetic(i, 1)
    y = lax.bitcast_convert_type(i, jnp.float32)
    for _ in range(3):
        y = y * (1.5 - 0.5 * x * y * y)
    return y


def _make_sc_kernel(n_pairs, n_pos):
    assert n_pairs % (NW * CHUNK) == 0
    ppt = n_pairs // NW
    nch = ppt // CHUNK
    mesh = plsc.VectorSubcoreMesh(core_axis_name="c", subcore_axis_name="s")

    @functools.partial(
        pl.kernel,
        mesh=mesh,
        compiler_params=pltpu.CompilerParams(
            needs_layout_passes=False, use_tc_tiling_on_sc=False),
        out_type=[
            jax.ShapeDtypeStruct((NW, L), jnp.float32),
            jax.ShapeDtypeStruct((NW, L), jnp.float32),
        ],
        scratch_types=[
            pltpu.VMEM((ppt,), jnp.int32),
            pltpu.VMEM((ppt,), jnp.int32),
            pltpu.VMEM((CHUNK, D), jnp.float32),
            pltpu.VMEM((CHUNK, D), jnp.float32),
            pltpu.VMEM((L,), jnp.float32),
            pltpu.VMEM((L,), jnp.float32),
            pltpu.SemaphoreType.DMA,
            pltpu.SemaphoreType.DMA,
        ],
    )
    def sc_kernel(z_hbm, ii_hbm, jj_hbm, oexp_hbm, opos_hbm,
                  ii_v, jj_v, rows_i, rows_j,
                  accexp, accpos, sem_i, sem_j):
        wid = lax.axis_index("s") * NC + lax.axis_index("c")
        base = wid * ppt
        pltpu.sync_copy(ii_hbm.at[pl.ds(base, ppt)], ii_v)
        pltpu.sync_copy(jj_hbm.at[pl.ds(base, ppt)], jj_v)
        accexp[...] = jnp.zeros((L,), jnp.float32)
        accpos[...] = jnp.zeros((L,), jnp.float32)
        lane = lax.broadcasted_iota(jnp.int32, (L,), 0)

        def chunk_body(ch, _):
            off = ch * CHUNK
            cp_i = pltpu.async_copy(
                z_hbm.at[ii_v.at[pl.ds(off, CHUNK)]], rows_i, sem_i)
            cp_j = pltpu.async_copy(
                z_hbm.at[jj_v.at[pl.ds(off, CHUNK)]], rows_j, sem_j)
            cp_i.wait()
            cp_j.wait()

            def group_body(g, _):
                ab_vec = jnp.zeros((L,), jnp.float32)
                aa_vec = jnp.zeros((L,), jnp.float32)
                bb_vec = jnp.zeros((L,), jnp.float32)
                for k in range(L):
                    p = g * L + k
                    ab = jnp.zeros((L,), jnp.float32)
                    aa = jnp.zeros((L,), jnp.float32)
                    bb = jnp.zeros((L,), jnp.float32)
                    for s in range(D // L):
                        av = rows_i[p, pl.ds(s * L, L)]
                        bv = rows_j[p, pl.ds(s * L, L)]
                        ab = ab + av * bv
                        aa = aa + av * av
                        bb = bb + bv * bv
                    ab_vec = jnp.where(lane == k, jnp.sum(ab), ab_vec)
                    aa_vec = jnp.where(lane == k, jnp.sum(aa), aa_vec)
                    bb_vec = jnp.where(lane == k, jnp.sum(bb), bb_vec)
                na = aa_vec * _fast_rsqrt(aa_vec)
                nb = bb_vec * _fast_rsqrt(bb_vec)
                denom = jnp.maximum(na, EPS) * jnp.maximum(nb, EPS)
                logit = (ab_vec / denom) * TEMP_INV
                accexp[...] += jnp.exp(logit - 10.0)
                gidx = base + off + g * L + lane
                accpos[...] += jnp.where(gidx < n_pos, logit, 0.0)
                return 0

            lax.fori_loop(0, CHUNK // L, group_body, 0)
            return 0

        lax.fori_loop(0, nch, chunk_body, 0)
        pltpu.sync_copy(accexp, oexp_hbm.at[wid])
        pltpu.sync_copy(accpos, opos_hbm.at[wid])

    return sc_kernel


def kernel(z_all, pos_pairs, neg_pairs):
    n_pos = pos_pairs.shape[0]
    pairs = jnp.concatenate([pos_pairs, neg_pairs], axis=0)
    ii = pairs[:, 0]
    jj = pairs[:, 1]
    sc = _make_sc_kernel(pairs.shape[0], n_pos)
    part_exp, part_pos = sc(z_all, ii, jj)
    lse = 10.0 + jnp.log(jnp.sum(part_exp))
    return lse - jnp.sum(part_pos) / n_pos
